# baseline (speedup 1.0000x reference)
import jax
import jax.numpy as jnp
from jax import lax
from jax.experimental import pallas as pl
from jax.experimental.pallas import tpu as pltpu
from jax.experimental.pallas import tpu_sc as plsc

BATCH = 16384
EMBED_DIM = 32
NUM_WORKERS = 32
B_PER_W = BATCH // NUM_WORKERS
CHUNK = 128
N_CHUNKS = B_PER_W // CHUNK
LANES = 16


def _body(uids_hbm, iids_hbm, utab_hbm, itab_hbm, out_hbm,
          uidx_v, iidx_v, u_v, i_v, out_v, sem):
    wid = lax.axis_index("s") * 2 + lax.axis_index("c")
    base = wid * B_PER_W

    pltpu.sync_copy(uids_hbm.at[pl.ds(base, B_PER_W)], uidx_v)
    pltpu.sync_copy(iids_hbm.at[pl.ds(base, B_PER_W)], iidx_v)

    def c_body(c, carry):
        copies = []
        for j in range(N_CHUNKS):
            sl = pl.ds(j * CHUNK, CHUNK)
            copies.append(pltpu.async_copy(
                utab_hbm.at[c].at[uidx_v.at[sl]], u_v.at[c, sl], sem))
            copies.append(pltpu.async_copy(
                itab_hbm.at[c].at[iidx_v.at[sl]], i_v.at[c, sl], sem))
        for cp in copies:
            cp.wait()
        return carry

    lax.fori_loop(0, EMBED_DIM, c_body, 0)

    iota16 = lax.iota(jnp.int32, LANES)

    def group_body(g, carry):
        sl = pl.ds(g * LANES, LANES)
        acc = jnp.zeros((LANES,), jnp.float32)
        for c in range(EMBED_DIM):
            acc = acc + u_v[c, sl] * i_v[c, sl]
        e = jnp.exp(-jnp.abs(acc))
        num = jnp.where(acc >= 0, jnp.ones_like(acc), e)
        plsc.store_scatter(out_v, [g * LANES + iota16], num / (1.0 + e))
        return carry

    lax.fori_loop(0, B_PER_W // LANES, group_body, 0)

    pltpu.sync_copy(out_v, out_hbm.at[pl.ds(base, B_PER_W)])


@jax.jit
def kernel(user_ids, item_ids, user_table, item_table):
    uids = user_ids.astype(jnp.int32)
    iids = item_ids.astype(jnp.int32)
    utab = user_table.T
    itab = item_table.T

    mesh = plsc.VectorSubcoreMesh(core_axis_name="c", subcore_axis_name="s")
    run = pl.kernel(
        _body, mesh=mesh,
        out_type=jax.ShapeDtypeStruct((BATCH,), jnp.float32),
        compiler_params=pltpu.CompilerParams(
            use_tc_tiling_on_sc=False, needs_layout_passes=False),
        scratch_types=[
            pltpu.VMEM((B_PER_W,), jnp.int32),
            pltpu.VMEM((B_PER_W,), jnp.int32),
            pltpu.VMEM((EMBED_DIM, B_PER_W), jnp.float32),
            pltpu.VMEM((EMBED_DIM, B_PER_W), jnp.float32),
            pltpu.VMEM((B_PER_W,), jnp.float32),
            pltpu.SemaphoreType.DMA,
        ],
    )
    return run(uids, iids, utab, itab)


# speedup vs baseline: 1.0040x; 1.0040x over previous
import jax
import jax.numpy as jnp
from jax import lax
from jax.experimental import pallas as pl
from jax.experimental.pallas import tpu as pltpu
from jax.experimental.pallas import tpu_sc as plsc

BATCH = 16384
EMBED_DIM = 32
NUM_WORKERS = 32
B_PER_W = BATCH // NUM_WORKERS
CHUNK = 128
N_CHUNKS = B_PER_W // CHUNK
LANES = 16


def _body(uids_hbm, iids_hbm, utab_hbm, itab_hbm, out_hbm,
          uidx_v, iidx_v, u_v, i_v, out_v, sem):
    wid = lax.axis_index("s") * 2 + lax.axis_index("c")
    base = wid * B_PER_W

    pltpu.sync_copy(uids_hbm.at[pl.ds(base, B_PER_W)], uidx_v)
    pltpu.sync_copy(iids_hbm.at[pl.ds(base, B_PER_W)], iidx_v)

    def c_body(c, carry):
        for j in range(N_CHUNKS):
            sl = pl.ds(j * CHUNK, CHUNK)
            pltpu.async_copy(
                utab_hbm.at[c].at[uidx_v.at[sl]], u_v.at[c, sl], sem)
            pltpu.async_copy(
                itab_hbm.at[c].at[iidx_v.at[sl]], i_v.at[c, sl], sem)
        return carry

    lax.fori_loop(0, EMBED_DIM, c_body, 0)
    pltpu.make_async_copy(
        utab_hbm.at[:, pl.ds(0, B_PER_W)], u_v, sem).wait()
    pltpu.make_async_copy(
        itab_hbm.at[:, pl.ds(0, B_PER_W)], i_v, sem).wait()

    iota16 = lax.iota(jnp.int32, LANES)

    def group_body(g, carry):
        sl = pl.ds(g * LANES, LANES)
        acc = jnp.zeros((LANES,), jnp.float32)
        for c in range(EMBED_DIM):
            acc = acc + u_v[c, sl] * i_v[c, sl]
        e = jnp.exp(-jnp.abs(acc))
        num = jnp.where(acc >= 0, jnp.ones_like(acc), e)
        plsc.store_scatter(out_v, [g * LANES + iota16], num / (1.0 + e))
        return carry

    lax.fori_loop(0, B_PER_W // LANES, group_body, 0)

    pltpu.sync_copy(out_v, out_hbm.at[pl.ds(base, B_PER_W)])


@jax.jit
def kernel(user_ids, item_ids, user_table, item_table):
    uids = user_ids.astype(jnp.int32)
    iids = item_ids.astype(jnp.int32)
    utab = user_table.T
    itab = item_table.T

    mesh = plsc.VectorSubcoreMesh(core_axis_name="c", subcore_axis_name="s")
    run = pl.kernel(
        _body, mesh=mesh,
        out_type=jax.ShapeDtypeStruct((BATCH,), jnp.float32),
        compiler_params=pltpu.CompilerParams(
            use_tc_tiling_on_sc=False, needs_layout_passes=False),
        scratch_types=[
            pltpu.VMEM((B_PER_W,), jnp.int32),
            pltpu.VMEM((B_PER_W,), jnp.int32),
            pltpu.VMEM((EMBED_DIM, B_PER_W), jnp.float32),
            pltpu.VMEM((EMBED_DIM, B_PER_W), jnp.float32),
            pltpu.VMEM((B_PER_W,), jnp.float32),
            pltpu.SemaphoreType.DMA,
        ],
    )
    return run(uids, iids, utab, itab)
